# initial kernel scaffold (unmeasured)
import functools

import jax
import jax.numpy as jnp
from jax import lax
from jax.experimental import pallas as pl
from jax.experimental.pallas import tpu as pltpu

N_DEV = 4
B, SQ, H, D = 16, 1, 16, 64
SCALE = D ** -0.5


def _local_partials(Q, K, V):
    kv_per = K.shape[1]

    def body(q_ref, k_ref, v_ref, u_ref, m_ref, l_ref):
        q = q_ref[0, 0].astype(jnp.bfloat16)
        k = k_ref[0].astype(jnp.bfloat16)
        v = v_ref[0].astype(jnp.bfloat16)
        s = lax.dot_general(
            q, k,
            dimension_numbers=(((1,), (2,)), ((0,), (1,))),
            preferred_element_type=jnp.float32,
        ) * SCALE
        m = jnp.max(s, axis=1)
        p = jnp.exp(s - m[:, None])
        l = jnp.sum(p, axis=1)
        u = lax.dot_general(
            p.astype(jnp.bfloat16), v,
            dimension_numbers=(((1,), (0,)), ((0,), (1,))),
            preferred_element_type=jnp.float32,
        )
        u_ref[0] = u
        m_ref[0, :] = m
        l_ref[0, :] = l

    return pl.pallas_call(
        body,
        grid=(B,),
        in_specs=[
            pl.BlockSpec((1, SQ, H, D), lambda b: (b, 0, 0, 0)),
            pl.BlockSpec((1, kv_per, H, D), lambda b: (b, 0, 0, 0)),
            pl.BlockSpec((1, kv_per, H, D), lambda b: (b, 0, 0, 0)),
        ],
        out_specs=[
            pl.BlockSpec((1, H, D), lambda b: (b, 0, 0)),
            pl.BlockSpec((1, H), lambda b: (b, 0)),
            pl.BlockSpec((1, H), lambda b: (b, 0)),
        ],
        out_shape=[
            jax.ShapeDtypeStruct((B, H, D), jnp.float32),
            jax.ShapeDtypeStruct((B, H), jnp.float32),
            jax.ShapeDtypeStruct((B, H), jnp.float32),
        ],
    )(Q, K, V)


def _ring_combine(U, m, l):

    def body(u_ref, m_ref, l_ref, out_ref, g_u, g_ml, send_u, recv_u,
             send_ml, recv_ml):
        my = lax.axis_index("i")
        left = lax.rem(my + N_DEV - 1, N_DEV)
        right = lax.rem(my + 1, N_DEV)

        barrier = pltpu.get_barrier_semaphore()
        for nbr in (left, right):
            pl.semaphore_signal(
                barrier, inc=1, device_id=(nbr,),
                device_id_type=pl.DeviceIdType.MESH,
            )
        pl.semaphore_wait(barrier, 2)

        g_u[pl.ds(my, 1)] = u_ref[...][None]
        g_ml[pl.ds(my, 1), 0] = m_ref[...][None]
        g_ml[pl.ds(my, 1), 1] = l_ref[...][None]

        for h in range(N_DEV - 1):
            slot = lax.rem(my + N_DEV - h, N_DEV)
            rdma_u = pltpu.make_async_remote_copy(
                src_ref=g_u.at[pl.ds(slot, 1)],
                dst_ref=g_u.at[pl.ds(slot, 1)],
                send_sem=send_u.at[h],
                recv_sem=recv_u.at[h],
                device_id=(right,),
                device_id_type=pl.DeviceIdType.MESH,
            )
            rdma_ml = pltpu.make_async_remote_copy(
                src_ref=g_ml.at[pl.ds(slot, 1)],
                dst_ref=g_ml.at[pl.ds(slot, 1)],
                send_sem=send_ml.at[h],
                recv_sem=recv_ml.at[h],
                device_id=(right,),
                device_id_type=pl.DeviceIdType.MESH,
            )
            rdma_u.start()
            rdma_ml.start()
            rdma_u.wait()
            rdma_ml.wait()

        m_all = g_ml[:, 0]
        l_all = g_ml[:, 1]
        m_g = jnp.max(m_all, axis=0)
        alpha = jnp.exp(m_all - m_g[None])
        l_g = jnp.sum(l_all * alpha, axis=0)
        u_g = jnp.sum(g_u[...] * alpha[..., None], axis=0)
        out_ref[...] = (u_g / l_g[..., None])[:, None]

    return pl.pallas_call(
        body,
        out_shape=jax.ShapeDtypeStruct((B, SQ, H, D), jnp.float32),
        in_specs=[pl.BlockSpec(memory_space=pltpu.VMEM)] * 3,
        out_specs=pl.BlockSpec(memory_space=pltpu.VMEM),
        scratch_shapes=[
            pltpu.VMEM((N_DEV, B, H, D), jnp.float32),
            pltpu.VMEM((N_DEV, 2, B, H), jnp.float32),
            pltpu.SemaphoreType.DMA((N_DEV - 1,)),
            pltpu.SemaphoreType.DMA((N_DEV - 1,)),
            pltpu.SemaphoreType.DMA((N_DEV - 1,)),
            pltpu.SemaphoreType.DMA((N_DEV - 1,)),
        ],
        compiler_params=pltpu.CompilerParams(collective_id=0),
    )(U, m, l)


def kernel(Q, K, V):
    U, m, l = _local_partials(Q, K, V)
    return _ring_combine(U, m, l)


# baseline (device time: 411393 ns/iter reference)
import functools

import jax
import jax.numpy as jnp
from jax import lax
from jax.experimental import pallas as pl
from jax.experimental.pallas import tpu as pltpu

N_DEV = 4
B, SQ, H, D = 16, 1, 16, 64
SCALE = D ** -0.5


def _local_partials(Q, K, V):
    kv_per = K.shape[1]

    def body(q_ref, k_ref, v_ref, u_ref, m_ref, l_ref):
        q = q_ref[0, 0].astype(jnp.bfloat16)
        k = k_ref[0].astype(jnp.bfloat16)
        v = v_ref[0].astype(jnp.bfloat16)
        s = lax.dot_general(
            q, k,
            dimension_numbers=(((1,), (2,)), ((0,), (1,))),
            preferred_element_type=jnp.float32,
        ) * SCALE
        m = jnp.max(s, axis=1)
        p = jnp.exp(s - m[:, None])
        l = jnp.sum(p, axis=1)
        u = lax.dot_general(
            p.astype(jnp.bfloat16), v,
            dimension_numbers=(((1,), (0,)), ((0,), (1,))),
            preferred_element_type=jnp.float32,
        )
        u_ref[0] = u
        m_ref[0, 0, :] = m
        l_ref[0, 0, :] = l

    return pl.pallas_call(
        body,
        grid=(B,),
        in_specs=[
            pl.BlockSpec((1, SQ, H, D), lambda b: (b, 0, 0, 0)),
            pl.BlockSpec((1, kv_per, H, D), lambda b: (b, 0, 0, 0)),
            pl.BlockSpec((1, kv_per, H, D), lambda b: (b, 0, 0, 0)),
        ],
        out_specs=[
            pl.BlockSpec((1, H, D), lambda b: (b, 0, 0)),
            pl.BlockSpec((1, 1, H), lambda b: (b, 0, 0)),
            pl.BlockSpec((1, 1, H), lambda b: (b, 0, 0)),
        ],
        out_shape=[
            jax.ShapeDtypeStruct((B, H, D), jnp.float32),
            jax.ShapeDtypeStruct((B, 1, H), jnp.float32),
            jax.ShapeDtypeStruct((B, 1, H), jnp.float32),
        ],
        compiler_params=pltpu.CompilerParams(
            vmem_limit_bytes=100 * 1024 * 1024,
        ),
    )(Q, K, V)


def _ring_combine(U, m, l):

    def body(u_ref, m_ref, l_ref, out_ref, g_u, g_ml, send_u, recv_u,
             send_ml, recv_ml):
        my = lax.axis_index("i")
        left = lax.rem(my + N_DEV - 1, N_DEV)
        right = lax.rem(my + 1, N_DEV)

        barrier = pltpu.get_barrier_semaphore()
        for nbr in (left, right):
            pl.semaphore_signal(
                barrier, inc=1, device_id=(nbr,),
                device_id_type=pl.DeviceIdType.MESH,
            )
        pl.semaphore_wait(barrier, 2)

        g_u[pl.ds(my, 1)] = u_ref[...][None]
        g_ml[pl.ds(my, 1), 0] = m_ref[:, 0, :][None]
        g_ml[pl.ds(my, 1), 1] = l_ref[:, 0, :][None]

        for h in range(N_DEV - 1):
            slot = lax.rem(my + N_DEV - h, N_DEV)
            rdma_u = pltpu.make_async_remote_copy(
                src_ref=g_u.at[pl.ds(slot, 1)],
                dst_ref=g_u.at[pl.ds(slot, 1)],
                send_sem=send_u.at[h],
                recv_sem=recv_u.at[h],
                device_id=(right,),
                device_id_type=pl.DeviceIdType.MESH,
            )
            rdma_ml = pltpu.make_async_remote_copy(
                src_ref=g_ml.at[pl.ds(slot, 1)],
                dst_ref=g_ml.at[pl.ds(slot, 1)],
                send_sem=send_ml.at[h],
                recv_sem=recv_ml.at[h],
                device_id=(right,),
                device_id_type=pl.DeviceIdType.MESH,
            )
            rdma_u.start()
            rdma_ml.start()
            rdma_u.wait()
            rdma_ml.wait()

        m_all = g_ml[:, 0]
        l_all = g_ml[:, 1]
        m_g = jnp.max(m_all, axis=0)
        alpha = jnp.exp(m_all - m_g[None])
        l_g = jnp.sum(l_all * alpha, axis=0)
        u_g = jnp.sum(g_u[...] * alpha[..., None], axis=0)
        out_ref[...] = (u_g / l_g[..., None])[:, None]

    return pl.pallas_call(
        body,
        out_shape=jax.ShapeDtypeStruct((B, SQ, H, D), jnp.float32),
        in_specs=[pl.BlockSpec(memory_space=pltpu.VMEM)] * 3,
        out_specs=pl.BlockSpec(memory_space=pltpu.VMEM),
        scratch_shapes=[
            pltpu.VMEM((N_DEV, B, H, D), jnp.float32),
            pltpu.VMEM((N_DEV, 2, B, H), jnp.float32),
            pltpu.SemaphoreType.DMA((N_DEV - 1,)),
            pltpu.SemaphoreType.DMA((N_DEV - 1,)),
            pltpu.SemaphoreType.DMA((N_DEV - 1,)),
            pltpu.SemaphoreType.DMA((N_DEV - 1,)),
        ],
        compiler_params=pltpu.CompilerParams(collective_id=0),
    )(U, m, l)


def kernel(Q, K, V):
    U, m, l = _local_partials(Q, K, V)
    return _ring_combine(U, m, l)


# device time: 323213 ns/iter; 1.2728x vs baseline; 1.2728x over previous
import functools

import jax
import jax.numpy as jnp
from jax import lax
from jax.experimental import pallas as pl
from jax.experimental.pallas import tpu as pltpu

N_DEV = 4
B, SQ, H, D = 16, 1, 16, 64
SCALE = D ** -0.5


def _local_partials(Q, K, V):
    kv_per = K.shape[1]

    def body(q_ref, k_ref, v_ref, u_ref, m_ref, l_ref):
        q = q_ref[0, 0]
        k = k_ref[0]
        v = v_ref[0]
        s = jnp.sum(k * q[None], axis=-1) * SCALE
        m = jnp.max(s, axis=0)
        p = jnp.exp(s - m[None])
        l = jnp.sum(p, axis=0)
        u = jnp.sum(v * p[:, :, None], axis=0)
        u_ref[0] = u
        m_ref[0, 0, :] = m
        l_ref[0, 0, :] = l

    return pl.pallas_call(
        body,
        grid=(B,),
        in_specs=[
            pl.BlockSpec((1, SQ, H, D), lambda b: (b, 0, 0, 0)),
            pl.BlockSpec((1, kv_per, H, D), lambda b: (b, 0, 0, 0)),
            pl.BlockSpec((1, kv_per, H, D), lambda b: (b, 0, 0, 0)),
        ],
        out_specs=[
            pl.BlockSpec((1, H, D), lambda b: (b, 0, 0)),
            pl.BlockSpec((1, 1, H), lambda b: (b, 0, 0)),
            pl.BlockSpec((1, 1, H), lambda b: (b, 0, 0)),
        ],
        out_shape=[
            jax.ShapeDtypeStruct((B, H, D), jnp.float32),
            jax.ShapeDtypeStruct((B, 1, H), jnp.float32),
            jax.ShapeDtypeStruct((B, 1, H), jnp.float32),
        ],
        compiler_params=pltpu.CompilerParams(
            vmem_limit_bytes=100 * 1024 * 1024,
        ),
    )(Q, K, V)


def _ring_combine(U, m, l):

    def body(u_ref, m_ref, l_ref, out_ref, g_u, g_ml, send_u, recv_u,
             send_ml, recv_ml):
        my = lax.axis_index("i")
        left = lax.rem(my + N_DEV - 1, N_DEV)
        right = lax.rem(my + 1, N_DEV)

        barrier = pltpu.get_barrier_semaphore()
        for nbr in (left, right):
            pl.semaphore_signal(
                barrier, inc=1, device_id=(nbr,),
                device_id_type=pl.DeviceIdType.MESH,
            )
        pl.semaphore_wait(barrier, 2)

        g_u[pl.ds(my, 1)] = u_ref[...][None]
        g_ml[pl.ds(my, 1), 0] = m_ref[:, 0, :][None]
        g_ml[pl.ds(my, 1), 1] = l_ref[:, 0, :][None]

        for h in range(N_DEV - 1):
            slot = lax.rem(my + N_DEV - h, N_DEV)
            rdma_u = pltpu.make_async_remote_copy(
                src_ref=g_u.at[pl.ds(slot, 1)],
                dst_ref=g_u.at[pl.ds(slot, 1)],
                send_sem=send_u.at[h],
                recv_sem=recv_u.at[h],
                device_id=(right,),
                device_id_type=pl.DeviceIdType.MESH,
            )
            rdma_ml = pltpu.make_async_remote_copy(
                src_ref=g_ml.at[pl.ds(slot, 1)],
                dst_ref=g_ml.at[pl.ds(slot, 1)],
                send_sem=send_ml.at[h],
                recv_sem=recv_ml.at[h],
                device_id=(right,),
                device_id_type=pl.DeviceIdType.MESH,
            )
            rdma_u.start()
            rdma_ml.start()
            rdma_u.wait()
            rdma_ml.wait()

        m_all = g_ml[:, 0]
        l_all = g_ml[:, 1]
        m_g = jnp.max(m_all, axis=0)
        alpha = jnp.exp(m_all - m_g[None])
        l_g = jnp.sum(l_all * alpha, axis=0)
        u_g = jnp.sum(g_u[...] * alpha[..., None], axis=0)
        out_ref[...] = (u_g / l_g[..., None])[:, None]

    return pl.pallas_call(
        body,
        out_shape=jax.ShapeDtypeStruct((B, SQ, H, D), jnp.float32),
        in_specs=[pl.BlockSpec(memory_space=pltpu.VMEM)] * 3,
        out_specs=pl.BlockSpec(memory_space=pltpu.VMEM),
        scratch_shapes=[
            pltpu.VMEM((N_DEV, B, H, D), jnp.float32),
            pltpu.VMEM((N_DEV, 2, B, H), jnp.float32),
            pltpu.SemaphoreType.DMA((N_DEV - 1,)),
            pltpu.SemaphoreType.DMA((N_DEV - 1,)),
            pltpu.SemaphoreType.DMA((N_DEV - 1,)),
            pltpu.SemaphoreType.DMA((N_DEV - 1,)),
        ],
        compiler_params=pltpu.CompilerParams(collective_id=0),
    )(U, m, l)


def kernel(Q, K, V):
    U, m, l = _local_partials(Q, K, V)
    return _ring_combine(U, m, l)
